# baseline (speedup 1.0000x reference)
import functools

import jax
import jax.numpy as jnp
from jax import lax
from jax.experimental import pallas as pl
from jax.experimental.pallas import tpu as pltpu
from jax.experimental.pallas import tpu_sc as plsc

_A = 3
_C = 80
_H = 64
_W = 64
_NS = 24
_K = 8
_Q = 8
_ROWS = _H // _Q
_POS = _ROWS * _W
_ANC_W = (10.0 / 32.0, 16.0 / 32.0, 33.0 / 32.0)
_ANC_H = (13.0 / 32.0, 30.0 / 32.0, 23.0 / 32.0)

_mesh = plsc.VectorSubcoreMesh(core_axis_name="c", subcore_axis_name="s")


@functools.partial(
    pl.kernel,
    mesh=_mesh,
    out_type=[
        jax.ShapeDtypeStruct((_NS, _Q, _POS), jnp.float32),
        jax.ShapeDtypeStruct((_NS, _Q, _POS), jnp.int32),
    ],
    scratch_types=[
        pltpu.VMEM((_C, _ROWS, _W), jnp.float32),
        pltpu.VMEM((_POS,), jnp.float32),
        pltpu.VMEM((_POS,), jnp.int32),
        pltpu.SemaphoreType.DMA,
    ],
)
def _sc_classred(in_hbm, cc_hbm, id_hbm, lbuf, cbuf, ibuf, sem):
    wid = lax.axis_index("s") * 2 + lax.axis_index("c")

    def do_chunk(q, _):
        slab = q // _Q
        chunk = lax.rem(q, _Q)
        b = slab // _A
        a = lax.rem(slab, _A)
        pltpu.async_copy(
            in_hbm.at[b, pl.ds(a * (5 + _C) + 5, _C),
                      pl.ds(chunk * _ROWS, _ROWS)],
            lbuf,
            sem,
        ).wait()

        def do_group(g, _):
            y = g // (_W // 16)
            x0 = lax.rem(g, _W // 16) * 16
            v0 = lbuf[0, y, pl.ds(x0, 16)]

            def do_class(c, carry):
                best, bidx, ssum = carry
                v = lbuf[c, y, pl.ds(x0, 16)]
                gt = v > best
                best = jnp.where(gt, v, best)
                bidx = jnp.where(gt, c, bidx)
                ssum = ssum + jnp.exp(v)
                return best, bidx, ssum

            best, bidx, ssum = lax.fori_loop(
                1, _C, do_class,
                (v0, jnp.zeros((16,), jnp.int32), jnp.exp(v0)))
            cbuf[pl.ds(g * 16, 16)] = jnp.exp(best) / ssum
            ibuf[pl.ds(g * 16, 16)] = bidx
            return 0

        lax.fori_loop(0, _POS // 16, do_group, 0)
        pltpu.sync_copy(cbuf, cc_hbm.at[slab, chunk])
        pltpu.sync_copy(ibuf, id_hbm.at[slab, chunk])
        return 0

    lax.fori_loop(wid * 6, wid * 6 + 6, do_chunk, 0)


def _tc_kernel(thr_ref, in_ref, bo_ref, mk_ref, buf_ref, sem_ref):
    def slab_copy(i, slot):
        b = i // _A
        a = i % _A
        return pltpu.make_async_copy(
            in_ref.at[b, pl.ds(a * (5 + _C), 5)],
            buf_ref.at[slot],
            sem_ref.at[slot],
        )

    for i in range(_K):
        slab_copy(i, i).start()

    gx = jax.lax.broadcasted_iota(jnp.int32, (_H, _W), 1).astype(jnp.float32)
    gy = jax.lax.broadcasted_iota(jnp.int32, (_H, _W), 0).astype(jnp.float32)
    inv_w = jnp.float32(1.0 / _W)
    inv_h = jnp.float32(1.0 / _H)
    thr = thr_ref[0]

    def body(i, _):
        slot = jax.lax.rem(i, _K)
        a = jax.lax.rem(i, _A)
        slab_copy(i, slot).wait()
        o = buf_ref[slot]

        aw = jnp.where(a == 0, _ANC_W[0],
                       jnp.where(a == 1, _ANC_W[1], _ANC_W[2]))
        ah = jnp.where(a == 0, _ANC_H[0],
                       jnp.where(a == 1, _ANC_H[1], _ANC_H[2]))

        det = jax.nn.sigmoid(o[4])
        bo_ref[i, 0] = (jax.nn.sigmoid(o[0]) + gx) * inv_w
        bo_ref[i, 1] = (jax.nn.sigmoid(o[1]) + gy) * inv_h
        bo_ref[i, 2] = jnp.exp(o[2]) * (aw * inv_w)
        bo_ref[i, 3] = jnp.exp(o[3]) * (ah * inv_h)
        bo_ref[i, 4] = det
        mk_ref[i] = det > thr

        @pl.when(i + _K < _NS)
        def _():
            slab_copy(i + _K, slot).start()

        return 0

    jax.lax.fori_loop(0, _NS, body, 0)


@jax.jit
def _decode(output, thr):
    cc, ids = _sc_classred(output)
    bo5, mk = pl.pallas_call(
        _tc_kernel,
        in_specs=[
            pl.BlockSpec(memory_space=pltpu.SMEM),
            pl.BlockSpec(memory_space=pl.ANY),
        ],
        out_specs=[
            pl.BlockSpec(memory_space=pltpu.VMEM),
            pl.BlockSpec(memory_space=pltpu.VMEM),
        ],
        out_shape=[
            jax.ShapeDtypeStruct((_NS, 5, _H, _W), jnp.float32),
            jax.ShapeDtypeStruct((_NS, _H, _W), jnp.bool_),
        ],
        scratch_shapes=[
            pltpu.VMEM((_K, 5, _H, _W), jnp.float32),
            pltpu.SemaphoreType.DMA((_K,)),
        ],
    )(thr, output)
    return bo5, cc, ids, mk


def kernel(output, nms_thresh):
    b, ch, h, w = output.shape
    thr = jnp.asarray(nms_thresh, dtype=jnp.float32).reshape(1)
    bo5, cc, ids, mk = _decode(output, thr)
    n = b * _A * h * w
    fields6 = jnp.concatenate(
        [bo5, cc.reshape(_NS, 1, _H, _W)], axis=1)
    boxes = jnp.transpose(fields6, (0, 2, 3, 1)).reshape(n, 6)
    cls_max_ids = ids.reshape(n)
    keep_mask = mk.reshape(n)
    return boxes, cls_max_ids, keep_mask


# speedup vs baseline: 1.2595x; 1.2595x over previous
import functools

import jax
import jax.numpy as jnp
from jax import lax
from jax.experimental import pallas as pl
from jax.experimental.pallas import tpu as pltpu
from jax.experimental.pallas import tpu_sc as plsc

_A = 3
_C = 80
_H = 64
_W = 64
_NS = 24
_K = 8
_Q = 8
_ROWS = _H // _Q
_POS = _ROWS * _W
_ANC_W = (10.0 / 32.0, 16.0 / 32.0, 33.0 / 32.0)
_ANC_H = (13.0 / 32.0, 30.0 / 32.0, 23.0 / 32.0)

_mesh = plsc.VectorSubcoreMesh(core_axis_name="c", subcore_axis_name="s")


@functools.partial(
    pl.kernel,
    mesh=_mesh,
    out_type=[
        jax.ShapeDtypeStruct((_NS, _Q, _POS), jnp.float32),
        jax.ShapeDtypeStruct((_NS, _Q, _POS), jnp.int32),
    ],
    scratch_types=[
        pltpu.VMEM((_C, _ROWS, _W), jnp.float32),
        pltpu.VMEM((_POS,), jnp.float32),
        pltpu.VMEM((_POS,), jnp.int32),
        pltpu.SemaphoreType.DMA,
    ],
)
def _sc_classred(in_hbm, cc_hbm, id_hbm, lbuf, cbuf, ibuf, sem):
    wid = lax.axis_index("s") * 2 + lax.axis_index("c")

    def do_chunk(q, _):
        slab = q // _Q
        chunk = lax.rem(q, _Q)
        b = slab // _A
        a = lax.rem(slab, _A)
        pltpu.async_copy(
            in_hbm.at[b, pl.ds(a * (5 + _C) + 5, _C),
                      pl.ds(chunk * _ROWS, _ROWS)],
            lbuf,
            sem,
        ).wait()

        def do_group(g, _):
            y = g // (_W // 16)
            x0 = lax.rem(g, _W // 16) * 16

            def do_class4(c4, carry):
                best, bidx, ssum = carry
                for k in range(4):
                    c = c4 * 4 + k
                    v = lbuf[c, y, pl.ds(x0, 16)]
                    gt = v > best
                    best = jnp.where(gt, v, best)
                    bidx = jnp.where(gt, c, bidx)
                    ssum = ssum + jnp.exp(v)
                return best, bidx, ssum

            neg_inf = jnp.full((16,), -jnp.inf, jnp.float32)
            best, bidx, ssum = lax.fori_loop(
                0, _C // 4, do_class4,
                (neg_inf, jnp.zeros((16,), jnp.int32),
                 jnp.zeros((16,), jnp.float32)))
            cbuf[pl.ds(g * 16, 16)] = jnp.exp(best) / ssum
            ibuf[pl.ds(g * 16, 16)] = bidx
            return 0

        lax.fori_loop(0, _POS // 16, do_group, 0)
        pltpu.sync_copy(cbuf, cc_hbm.at[slab, chunk])
        pltpu.sync_copy(ibuf, id_hbm.at[slab, chunk])
        return 0

    lax.fori_loop(wid * 6, wid * 6 + 6, do_chunk, 0)


def _tc_kernel(thr_ref, in_ref, bo_ref, mk_ref, buf_ref, sem_ref):
    def slab_copy(i, slot):
        b = i // _A
        a = i % _A
        return pltpu.make_async_copy(
            in_ref.at[b, pl.ds(a * (5 + _C), 5)],
            buf_ref.at[slot],
            sem_ref.at[slot],
        )

    for i in range(_K):
        slab_copy(i, i).start()

    gx = jax.lax.broadcasted_iota(jnp.int32, (_H, _W), 1).astype(jnp.float32)
    gy = jax.lax.broadcasted_iota(jnp.int32, (_H, _W), 0).astype(jnp.float32)
    inv_w = jnp.float32(1.0 / _W)
    inv_h = jnp.float32(1.0 / _H)
    thr = thr_ref[0]

    def body(i, _):
        slot = jax.lax.rem(i, _K)
        a = jax.lax.rem(i, _A)
        slab_copy(i, slot).wait()
        o = buf_ref[slot]

        aw = jnp.where(a == 0, _ANC_W[0],
                       jnp.where(a == 1, _ANC_W[1], _ANC_W[2]))
        ah = jnp.where(a == 0, _ANC_H[0],
                       jnp.where(a == 1, _ANC_H[1], _ANC_H[2]))

        det = jax.nn.sigmoid(o[4])
        bo_ref[i, 0] = (jax.nn.sigmoid(o[0]) + gx) * inv_w
        bo_ref[i, 1] = (jax.nn.sigmoid(o[1]) + gy) * inv_h
        bo_ref[i, 2] = jnp.exp(o[2]) * (aw * inv_w)
        bo_ref[i, 3] = jnp.exp(o[3]) * (ah * inv_h)
        bo_ref[i, 4] = det
        mk_ref[i] = det > thr

        @pl.when(i + _K < _NS)
        def _():
            slab_copy(i + _K, slot).start()

        return 0

    jax.lax.fori_loop(0, _NS, body, 0)


@jax.jit
def _decode(output, thr):
    cc, ids = _sc_classred(output)
    bo5, mk = pl.pallas_call(
        _tc_kernel,
        in_specs=[
            pl.BlockSpec(memory_space=pltpu.SMEM),
            pl.BlockSpec(memory_space=pl.ANY),
        ],
        out_specs=[
            pl.BlockSpec(memory_space=pltpu.VMEM),
            pl.BlockSpec(memory_space=pltpu.VMEM),
        ],
        out_shape=[
            jax.ShapeDtypeStruct((_NS, 5, _H, _W), jnp.float32),
            jax.ShapeDtypeStruct((_NS, _H, _W), jnp.bool_),
        ],
        scratch_shapes=[
            pltpu.VMEM((_K, 5, _H, _W), jnp.float32),
            pltpu.SemaphoreType.DMA((_K,)),
        ],
    )(thr, output)
    return bo5, cc, ids, mk


def kernel(output, nms_thresh):
    b, ch, h, w = output.shape
    thr = jnp.asarray(nms_thresh, dtype=jnp.float32).reshape(1)
    bo5, cc, ids, mk = _decode(output, thr)
    n = b * _A * h * w
    fields6 = jnp.concatenate(
        [bo5, cc.reshape(_NS, 1, _H, _W)], axis=1)
    boxes = jnp.transpose(fields6, (0, 2, 3, 1)).reshape(n, 6)
    cls_max_ids = ids.reshape(n)
    keep_mask = mk.reshape(n)
    return boxes, cls_max_ids, keep_mask


# speedup vs baseline: 1.2915x; 1.0254x over previous
import functools

import jax
import jax.numpy as jnp
from jax import lax
from jax.experimental import pallas as pl
from jax.experimental.pallas import tpu as pltpu
from jax.experimental.pallas import tpu_sc as plsc

_A = 3
_C = 80
_H = 64
_W = 64
_NS = 24
_K = 8
_Q = 8
_ROWS = _H // _Q
_POS = _ROWS * _W
_ANC_W = (10.0 / 32.0, 16.0 / 32.0, 33.0 / 32.0)
_ANC_H = (13.0 / 32.0, 30.0 / 32.0, 23.0 / 32.0)

_mesh = plsc.VectorSubcoreMesh(core_axis_name="c", subcore_axis_name="s")


@functools.partial(
    pl.kernel,
    mesh=_mesh,
    out_type=[
        jax.ShapeDtypeStruct((_NS, _Q, _POS), jnp.float32),
        jax.ShapeDtypeStruct((_NS, _Q, _POS), jnp.int32),
    ],
    scratch_types=[
        pltpu.VMEM((_C, _ROWS, _W), jnp.float32),
        pltpu.VMEM((_POS,), jnp.float32),
        pltpu.VMEM((_POS,), jnp.int32),
        pltpu.SemaphoreType.DMA,
    ],
)
def _sc_classred(in_hbm, cc_hbm, id_hbm, lbuf, cbuf, ibuf, sem):
    wid = lax.axis_index("s") * 2 + lax.axis_index("c")

    def do_chunk(q, _):
        slab = q // _Q
        chunk = lax.rem(q, _Q)
        b = slab // _A
        a = lax.rem(slab, _A)
        pltpu.async_copy(
            in_hbm.at[b, pl.ds(a * (5 + _C) + 5, _C),
                      pl.ds(chunk * _ROWS, _ROWS)],
            lbuf,
            sem,
        ).wait()

        def do_row(y, _):
            def do_class4(c4, carry):
                bests, bidxs, ssums = carry
                for k in range(4):
                    c = c4 * 4 + k
                    for xg in range(4):
                        v = lbuf[c, y, pl.ds(xg * 16, 16)]
                        gt = v > bests[xg]
                        bests[xg] = jnp.where(gt, v, bests[xg])
                        bidxs[xg] = jnp.where(gt, c, bidxs[xg])
                        ssums[xg] = ssums[xg] + jnp.exp(v)
                return bests, bidxs, ssums

            neg_inf = jnp.full((16,), -jnp.inf, jnp.float32)
            zf = jnp.zeros((16,), jnp.float32)
            zi = jnp.zeros((16,), jnp.int32)
            bests, bidxs, ssums = lax.fori_loop(
                0, _C // 4, do_class4,
                ([neg_inf] * 4, [zi] * 4, [zf] * 4))
            for xg in range(4):
                cbuf[pl.ds(y * _W + xg * 16, 16)] = (
                    jnp.exp(bests[xg]) / ssums[xg])
                ibuf[pl.ds(y * _W + xg * 16, 16)] = bidxs[xg]
            return 0

        lax.fori_loop(0, _ROWS, do_row, 0)
        pltpu.sync_copy(cbuf, cc_hbm.at[slab, chunk])
        pltpu.sync_copy(ibuf, id_hbm.at[slab, chunk])
        return 0

    lax.fori_loop(wid * 6, wid * 6 + 6, do_chunk, 0)


def _tc_kernel(thr_ref, in_ref, bo_ref, mk_ref, buf_ref, sem_ref):
    def slab_copy(i, slot):
        b = i // _A
        a = i % _A
        return pltpu.make_async_copy(
            in_ref.at[b, pl.ds(a * (5 + _C), 5)],
            buf_ref.at[slot],
            sem_ref.at[slot],
        )

    for i in range(_K):
        slab_copy(i, i).start()

    gx = jax.lax.broadcasted_iota(jnp.int32, (_H, _W), 1).astype(jnp.float32)
    gy = jax.lax.broadcasted_iota(jnp.int32, (_H, _W), 0).astype(jnp.float32)
    inv_w = jnp.float32(1.0 / _W)
    inv_h = jnp.float32(1.0 / _H)
    thr = thr_ref[0]

    def body(i, _):
        slot = jax.lax.rem(i, _K)
        a = jax.lax.rem(i, _A)
        slab_copy(i, slot).wait()
        o = buf_ref[slot]

        aw = jnp.where(a == 0, _ANC_W[0],
                       jnp.where(a == 1, _ANC_W[1], _ANC_W[2]))
        ah = jnp.where(a == 0, _ANC_H[0],
                       jnp.where(a == 1, _ANC_H[1], _ANC_H[2]))

        det = jax.nn.sigmoid(o[4])
        bo_ref[i, 0] = (jax.nn.sigmoid(o[0]) + gx) * inv_w
        bo_ref[i, 1] = (jax.nn.sigmoid(o[1]) + gy) * inv_h
        bo_ref[i, 2] = jnp.exp(o[2]) * (aw * inv_w)
        bo_ref[i, 3] = jnp.exp(o[3]) * (ah * inv_h)
        bo_ref[i, 4] = det
        mk_ref[i] = det > thr

        @pl.when(i + _K < _NS)
        def _():
            slab_copy(i + _K, slot).start()

        return 0

    jax.lax.fori_loop(0, _NS, body, 0)


@jax.jit
def _decode(output, thr):
    cc, ids = _sc_classred(output)
    bo5, mk = pl.pallas_call(
        _tc_kernel,
        in_specs=[
            pl.BlockSpec(memory_space=pltpu.SMEM),
            pl.BlockSpec(memory_space=pl.ANY),
        ],
        out_specs=[
            pl.BlockSpec(memory_space=pltpu.VMEM),
            pl.BlockSpec(memory_space=pltpu.VMEM),
        ],
        out_shape=[
            jax.ShapeDtypeStruct((_NS, 5, _H, _W), jnp.float32),
            jax.ShapeDtypeStruct((_NS, _H, _W), jnp.bool_),
        ],
        scratch_shapes=[
            pltpu.VMEM((_K, 5, _H, _W), jnp.float32),
            pltpu.SemaphoreType.DMA((_K,)),
        ],
    )(thr, output)
    return bo5, cc, ids, mk


def kernel(output, nms_thresh):
    b, ch, h, w = output.shape
    thr = jnp.asarray(nms_thresh, dtype=jnp.float32).reshape(1)
    bo5, cc, ids, mk = _decode(output, thr)
    n = b * _A * h * w
    fields6 = jnp.concatenate(
        [bo5, cc.reshape(_NS, 1, _H, _W)], axis=1)
    boxes = jnp.transpose(fields6, (0, 2, 3, 1)).reshape(n, 6)
    cls_max_ids = ids.reshape(n)
    keep_mask = mk.reshape(n)
    return boxes, cls_max_ids, keep_mask
